# baseline (speedup 1.0000x reference)
import jax
import jax.numpy as jnp
from jax.experimental import pallas as pl
from jax.experimental.pallas import tpu as pltpu

N, D, E, H = 2048, 1024, 8, 4096
BLK = 256
HB = 1024
NB = N // BLK
NH = H // HB


def _ffn_kernel(c_ref, x_ref, wg_ref, wv_ref, wo_ref, coef_ref, scale_ref, y_ref):
    e = pl.program_id(0)
    h = pl.program_id(1)
    j = pl.program_id(2)
    cnt = c_ref[e]

    @pl.when(j * BLK < cnt)
    def _active():
        x = x_ref[...]
        wg = wg_ref[0]
        wv = wv_ref[0]
        wo = wo_ref[0]

        gate = jax.lax.dot_general(x, wg, (((1,), (1,)), ((), ())),
                                   preferred_element_type=jnp.float32)
        gate = gate * 0.5 * (1.0 + jax.lax.erf(gate * 0.7071067811865476))
        value = jax.lax.dot_general(x, wv, (((1,), (1,)), ((), ())),
                                    preferred_element_type=jnp.float32)
        hidden = gate * value
        part = jax.lax.dot_general(hidden, wo, (((1,), (1,)), ((), ())),
                                   preferred_element_type=jnp.float32)

        srow = jax.lax.broadcasted_iota(jnp.int32, (1, E), 1) == e
        scale_e = jnp.sum(jnp.where(srow, scale_ref[...], 0.0))
        contrib = part * (coef_ref[...] * scale_e)

        rows = pl.ds(j * BLK, BLK)

        @pl.when(h == 0)
        def _init():
            y_ref[rows, :] = contrib

        @pl.when(h != 0)
        def _acc():
            y_ref[rows, :] += contrib


def _jmax(c):
    return jnp.maximum((c + BLK - 1) // BLK - 1, 0)


def _ragged_ffn(counts, Xg, coefc, Wg, Wv, Wo, scale):
    return pl.pallas_call(
        _ffn_kernel,
        grid_spec=pltpu.PrefetchScalarGridSpec(
            num_scalar_prefetch=1,
            grid=(E, NH, NB),
            in_specs=[
                pl.BlockSpec((BLK, D),
                             lambda e, h, j, c: (e * NB + jnp.minimum(j, _jmax(c[e])), 0)),
                pl.BlockSpec((1, HB, D), lambda e, h, j, c: (e, h, 0)),
                pl.BlockSpec((1, HB, D), lambda e, h, j, c: (e, h, 0)),
                pl.BlockSpec((1, D, HB), lambda e, h, j, c: (e, 0, h)),
                pl.BlockSpec((BLK, 1),
                             lambda e, h, j, c: (e * NB + jnp.minimum(j, _jmax(c[e])), 0)),
                pl.BlockSpec((1, E), lambda e, h, j, c: (0, 0)),
            ],
            out_specs=pl.BlockSpec((N, D), lambda e, h, j, c: (e, 0)),
        ),
        out_shape=jax.ShapeDtypeStruct((E * N, D), jnp.float32),
    )(counts, Xg, Wg, Wv, Wo, coefc, scale.reshape(1, E))


def kernel(tokens, dispatch_weights, combine_weights, Wg, Wv, Wo, scale):
    b, n, d = tokens.shape
    flat = tokens.reshape(n, d)
    disp = dispatch_weights.reshape(n, E)
    comb = combine_weights.reshape(n, E)

    mask = disp > 0
    counts = jnp.sum(mask.astype(jnp.int32), axis=0)
    perm = jnp.argsort(~mask, axis=0, stable=True)
    idx = perm.T
    kpos = jax.lax.broadcasted_iota(jnp.int32, (E, N), 1)
    valid = kpos < counts[:, None]
    combT = comb.T
    coefc = jnp.where(valid, jnp.take_along_axis(combT, idx, axis=1), 0.0)
    Xg = flat[idx.reshape(-1)]

    Y = _ragged_ffn(counts, Xg, coefc.reshape(E * N, 1), Wg, Wv, Wo, scale)

    Ysafe = jnp.where(valid.reshape(E * N, 1), Y, 0.0)
    out = jnp.zeros((n, d), jnp.float32).at[idx.reshape(-1)].add(Ysafe)
    return out.reshape(b, n, d)
